# Initial kernel scaffold; baseline (speedup 1.0000x reference)
import jax
import jax.numpy as jnp
from jax.experimental import pallas as pl


def kernel(qkv, positions, kv_cache, slot_mapping, q_weight, k_weight):
    raise NotImplementedError("write your pallas kernel here")


# speedup vs baseline: 4.3483x; 4.3483x over previous
import functools

import numpy as np
import jax
import jax.numpy as jnp
from jax import lax
from jax.experimental import pallas as pl
from jax.experimental.pallas import tpu as pltpu
from jax.experimental.pallas import tpu_sc as plsc

NUM_HEADS = 32
NUM_KV_HEADS = 8
HEAD = 128
EPS = 1e-06
BASE = 10000.0
T = 8192
NUM_SLOTS = 16384
Q_SIZE = NUM_HEADS * HEAD
KV_SIZE = NUM_KV_HEADS * HEAD
HALF = HEAD // 2

_INV_FREQ = (1.0 / (BASE ** (np.arange(0, HEAD, 2, dtype=np.float32) / HEAD))
             ).reshape(1, HALF)

BT = 256

NW = 32
TPW = T // NW
CH = 32
NCH = TPW // CH


def _tc_body(qkv_ref, pos_ref, qw_ref, kw_ref, q_ref, k_ref, v_ref):
    pos = pos_ref[...].astype(jnp.float32)
    j2d = lax.broadcasted_iota(jnp.int32, (1, HALF), 1).astype(jnp.float32)
    inv_freq = jnp.exp(j2d * np.float32(-2.0 * np.log(BASE) / HEAD))
    freqs = pos * inv_freq
    cos = jnp.cos(freqs)
    sin = jnp.sin(freqs)
    qw = qw_ref[...]
    kw = kw_ref[...]

    def norm_rope(x, w):
        var = jnp.mean(x * x, axis=1, keepdims=True)
        xn = x * lax.rsqrt(var + EPS) * w
        x1 = xn[:, :HALF]
        x2 = xn[:, HALF:]
        return jnp.concatenate([x1 * cos - x2 * sin, x2 * cos + x1 * sin],
                               axis=1)

    for h in range(NUM_HEADS):
        q_ref[:, h * HEAD:(h + 1) * HEAD] = norm_rope(
            qkv_ref[:, h * HEAD:(h + 1) * HEAD], qw)
    for h in range(NUM_KV_HEADS):
        k_ref[:, h * HEAD:(h + 1) * HEAD] = norm_rope(
            qkv_ref[:, Q_SIZE + h * HEAD:Q_SIZE + (h + 1) * HEAD], kw)
    v_ref[...] = qkv_ref[:, Q_SIZE + KV_SIZE:Q_SIZE + 2 * KV_SIZE]


def _tc_norm_rope(qkv, positions, q_weight, k_weight):
    return pl.pallas_call(
        _tc_body,
        grid=(T // BT,),
        in_specs=[
            pl.BlockSpec((BT, Q_SIZE + 2 * KV_SIZE), lambda i: (i, 0)),
            pl.BlockSpec((BT, 1), lambda i: (i, 0)),
            pl.BlockSpec((1, HEAD), lambda i: (0, 0)),
            pl.BlockSpec((1, HEAD), lambda i: (0, 0)),
        ],
        out_specs=[
            pl.BlockSpec((BT, Q_SIZE), lambda i: (i, 0)),
            pl.BlockSpec((BT, KV_SIZE), lambda i: (i, 0)),
            pl.BlockSpec((BT, KV_SIZE), lambda i: (i, 0)),
        ],
        out_shape=[
            jax.ShapeDtypeStruct((T, Q_SIZE), jnp.float32),
            jax.ShapeDtypeStruct((T, KV_SIZE), jnp.float32),
            jax.ShapeDtypeStruct((T, KV_SIZE), jnp.float32),
        ],
    )(qkv, positions.reshape(T, 1), q_weight.reshape(1, HEAD),
      k_weight.reshape(1, HEAD))


def _sc_scatter_body(k_hbm, v_hbm, slots_hbm, cache_hbm, slots_v, table_v,
                     widx, dstk, dstv, kbuf, vbuf, semk, semv):
    wid = lax.axis_index("s") * 2 + lax.axis_index("c")
    pltpu.sync_copy(slots_hbm, slots_v)
    iota = lax.iota(jnp.int32, 16)

    def build(i, carry):
        sl = slots_v[pl.ds(i * 16, 16)]
        _, last = plsc.scan_count(sl)
        plsc.store_scatter(table_v, [sl], iota + i * 16, mask=last)
        return carry

    lax.fori_loop(0, T // 16, build, 0)

    base = wid * TPW
    for j in range(TPW // 16):
        sl = slots_v[pl.ds(base + j * 16, 16)]
        w = plsc.load_gather(table_v, [sl])
        c = j // (CH // 16)
        off = (j % (CH // 16)) * 16
        widx[c, pl.ds(off, 16)] = w
        dstk[c, pl.ds(off, 16)] = sl
        dstv[c, pl.ds(off, 16)] = sl + NUM_SLOTS

    for c in range(NCH):
        pltpu.async_copy(k_hbm.at[widx.at[c]], kbuf, semk).wait()
        pltpu.async_copy(kbuf, cache_hbm.at[dstk.at[c]], semk).wait()
        pltpu.async_copy(v_hbm.at[widx.at[c]], vbuf, semv).wait()
        pltpu.async_copy(vbuf, cache_hbm.at[dstv.at[c]], semv).wait()


@functools.lru_cache(maxsize=1)
def _get_sc_scatter():
    mesh = plsc.VectorSubcoreMesh(core_axis_name="c", subcore_axis_name="s")
    return pl.kernel(
        _sc_scatter_body,
        mesh=mesh,
        out_type=(),
        compiler_params=pltpu.CompilerParams(needs_layout_passes=False),
        scratch_types=[
            pltpu.VMEM((T,), jnp.int32),
            pltpu.VMEM((NUM_SLOTS,), jnp.int32),
            pltpu.VMEM((NCH, CH), jnp.int32),
            pltpu.VMEM((NCH, CH), jnp.int32),
            pltpu.VMEM((NCH, CH), jnp.int32),
            pltpu.VMEM((CH, KV_SIZE), jnp.float32),
            pltpu.VMEM((CH, KV_SIZE), jnp.float32),
            pltpu.SemaphoreType.DMA,
            pltpu.SemaphoreType.DMA,
        ],
    )


def kernel(qkv, positions, kv_cache, slot_mapping, q_weight, k_weight):
    q2d, k2d, v2d = _tc_norm_rope(qkv, positions, q_weight, k_weight)
    cache_ref = jax.new_ref(kv_cache.reshape(2 * NUM_SLOTS, KV_SIZE))
    _get_sc_scatter()(k2d, v2d, slot_mapping, cache_ref)
    cache_new = jax.freeze(cache_ref).reshape(2, NUM_SLOTS, NUM_KV_HEADS, HEAD)
    return (q2d.reshape(T, NUM_HEADS, HEAD),
            k2d.reshape(T, NUM_KV_HEADS, HEAD),
            v2d.reshape(T, NUM_KV_HEADS, HEAD),
            cache_new)
